# baseline (speedup 1.0000x reference)
import functools

import jax
import jax.numpy as jnp
from jax import lax
from jax.experimental import pallas as pl
from jax.experimental.pallas import tpu as pltpu
from jax.experimental.pallas import tpu_sc as plsc

_B, _L, _D = 4096, 200, 32
_CH = 128


def _make_kernel():
    mesh = plsc.VectorSubcoreMesh(core_axis_name="c", subcore_axis_name="s")
    nc, ns = mesh.num_cores, mesh.num_subcores
    nw = nc * ns
    assert _B // _CH == nw

    @functools.partial(
        pl.kernel,
        out_type=jax.ShapeDtypeStruct((_L, _D // 8, nw, 8, _CH), jnp.float32),
        mesh=mesh,
        compiler_params=pltpu.CompilerParams(use_tc_tiling_on_sc=False,
                                             needs_layout_passes=False),
        scratch_types=[
            pltpu.VMEM((_L, _CH), jnp.int32),
            pltpu.VMEM((_L, _D), jnp.float32),
            pltpu.VMEM((_CH, _D), jnp.float32),
            pltpu.VMEM((_CH, _D), jnp.float32),
            pltpu.VMEM((_D // 8, 8, _CH + 1), jnp.float32),
            pltpu.VMEM((_D // 8, 8, _CH + 1), jnp.float32),
            pltpu.SemaphoreType.DMA,
            pltpu.SemaphoreType.DMA,
            pltpu.SemaphoreType.DMA,
            pltpu.SemaphoreType.DMA,
        ],
    )
    def emb_kernel(tok_hbm, xt_hbm, pos_hbm, out_hbm,
                   idx_v, pos_v, gbuf0, gbuf1, tbuf0, tbuf1,
                   gsem0, gsem1, ssem0, ssem1):
        wid = lax.axis_index("s") * nc + lax.axis_index("c")

        pltpu.sync_copy(xt_hbm.at[:, pl.ds(wid * _CH, _CH)], idx_v)
        pltpu.sync_copy(pos_hbm, pos_v)

        pltpu.async_copy(tok_hbm.at[idx_v.at[0]], gbuf0, gsem0)

        lane = lax.iota(jnp.int32, 16)
        zero = lane * 0
        lo0, lo1 = lane >> 3, lane & 7
        hi = lane + 16
        hi0, hi1 = hi >> 3, hi & 7

        def step(g, gbuf_b, gsem_b, ssem_b, tbuf_b, gbuf_n, gsem_n, ssem_n,
                 tbuf_n):
            @pl.when(g >= 1)
            def _():
                pltpu.make_async_copy(
                    tbuf_n.at[:, :, pl.ds(0, _CH)],
                    out_hbm.at[g - 1, :, wid], ssem_n).wait()

            @pl.when(g + 1 < _L)
            def _():
                pltpu.async_copy(tok_hbm.at[idx_v.at[g + 1]], gbuf_n, gsem_n)

            pltpu.make_async_copy(tok_hbm.at[idx_v.at[g]], gbuf_b, gsem_b).wait()

            pos_lo = pos_v[g, pl.ds(0, 16)]
            pos_hi = pos_v[g, pl.ds(16, 16)]
            for b0 in range(0, _CH, 8):
                vlo = [gbuf_b[b0 + k, pl.ds(0, 16)] + pos_lo for k in range(8)]
                vhi = [gbuf_b[b0 + k, pl.ds(16, 16)] + pos_hi for k in range(8)]
                for k in range(8):
                    cb = zero + (b0 + k)
                    plsc.store_scatter(tbuf_b, [lo0, lo1, cb], vlo[k])
                    plsc.store_scatter(tbuf_b, [hi0, hi1, cb], vhi[k])

            pltpu.async_copy(tbuf_b.at[:, :, pl.ds(0, _CH)],
                             out_hbm.at[g, :, wid], ssem_b)

        def outer(i, carry):
            g = i * 2
            step(g, gbuf0, gsem0, ssem0, tbuf0, gbuf1, gsem1, ssem1, tbuf1)
            step(g + 1, gbuf1, gsem1, ssem1, tbuf1, gbuf0, gsem0, ssem0, tbuf0)
            return carry

        lax.fori_loop(0, _L // 2, outer, 0)

        pltpu.make_async_copy(tbuf1.at[:, :, pl.ds(0, _CH)],
                              out_hbm.at[_L - 1, :, wid], ssem1).wait()

    return emb_kernel, nw


_V = 1000000
_TC = 800
_NCH = _V // _TC


def _make_transpose_kernel():
    mesh = plsc.VectorSubcoreMesh(core_axis_name="c", subcore_axis_name="s")
    nc, ns = mesh.num_cores, mesh.num_subcores
    nw = nc * ns
    tmax = (_NCH + nw - 1) // nw

    @functools.partial(
        pl.kernel,
        out_type=jax.ShapeDtypeStruct((_V, _D), jnp.float32),
        mesh=mesh,
        compiler_params=pltpu.CompilerParams(use_tc_tiling_on_sc=False,
                                             needs_layout_passes=False),
        scratch_types=[
            pltpu.VMEM((_D, _TC), jnp.float32),
            pltpu.VMEM((_D, _TC), jnp.float32),
            pltpu.VMEM((_TC, _D + 1), jnp.float32),
            pltpu.VMEM((_TC, _D + 1), jnp.float32),
            pltpu.SemaphoreType.DMA,
            pltpu.SemaphoreType.DMA,
            pltpu.SemaphoreType.DMA,
            pltpu.SemaphoreType.DMA,
        ],
    )
    def trans_kernel(tblT_hbm, out_hbm, ib0, ib1, ob0, ob1,
                     isem0, isem1, osem0, osem1):
        wid = lax.axis_index("s") * nc + lax.axis_index("c")
        lane = lax.iota(jnp.int32, 16)
        zero = lane * 0

        pltpu.async_copy(tblT_hbm.at[:, pl.ds(wid * _TC, _TC)], ib0, isem0)

        def step(t, ib_b, isem_b, ob_b, osem_b, ib_n, isem_n, ob_n, osem_n):
            k = wid + nw * t

            @pl.when(k < _NCH)
            def _():
                @pl.when(t >= 1)
                def _():
                    pltpu.make_async_copy(
                        ob_n.at[:, pl.ds(0, _D)],
                        out_hbm.at[pl.ds((k - nw) * _TC, _TC)], osem_n).wait()

                @pl.when(k + nw < _NCH)
                def _():
                    pltpu.async_copy(
                        tblT_hbm.at[:, pl.ds((k + nw) * _TC, _TC)],
                        ib_n, isem_n)

                pltpu.make_async_copy(
                    tblT_hbm.at[:, pl.ds(k * _TC, _TC)], ib_b, isem_b).wait()

                def body(vb, carry):
                    row = lane + vb * 16
                    for d in range(_D):
                        v = ib_b[d, pl.ds(vb * 16, 16)]
                        plsc.store_scatter(ob_b, [row, zero + d], v)
                    return carry

                lax.fori_loop(0, _TC // 16, body, 0)

                pltpu.async_copy(ob_b.at[:, pl.ds(0, _D)],
                                 out_hbm.at[pl.ds(k * _TC, _TC)], osem_b)

        def outer(i, carry):
            t = i * 2
            step(t, ib0, isem0, ob0, osem0, ib1, isem1, ob1, osem1)
            step(t + 1, ib1, isem1, ob1, osem1, ib0, isem0, ob0, osem0)
            return carry

        lax.fori_loop(0, tmax // 2, outer, 0)

        last_t = (_NCH - 1 - wid) // nw
        last_k = wid + nw * last_t

        @pl.when((last_t & 1) == 0)
        def _():
            pltpu.make_async_copy(
                ob0.at[:, pl.ds(0, _D)],
                out_hbm.at[pl.ds(last_k * _TC, _TC)], osem0).wait()

        @pl.when((last_t & 1) == 1)
        def _():
            pltpu.make_async_copy(
                ob1.at[:, pl.ds(0, _D)],
                out_hbm.at[pl.ds(last_k * _TC, _TC)], osem1).wait()

    return trans_kernel


def kernel(x, token_table, pos_table):
    emb, nw = _make_kernel()
    trans = _make_transpose_kernel()
    tblT = jnp.transpose(token_table)
    tbl_lin = trans(tblT)
    xt = jnp.transpose(x.astype(jnp.int32))
    out5 = emb(tbl_lin, xt, pos_table)
    out = jnp.transpose(out5, (2, 4, 0, 1, 3))
    return out.reshape(_B, _L, _D)


# speedup vs baseline: 5.2241x; 5.2241x over previous
import functools

import jax
import jax.numpy as jnp
from jax import lax
from jax.experimental import pallas as pl
from jax.experimental.pallas import tpu as pltpu
from jax.experimental.pallas import tpu_sc as plsc

_B, _L, _D = 4096, 200, 32
_CH = 128


def _make_kernel():
    mesh = plsc.VectorSubcoreMesh(core_axis_name="c", subcore_axis_name="s")
    nc, ns = mesh.num_cores, mesh.num_subcores
    nw = nc * ns
    assert _B // _CH == nw

    @functools.partial(
        pl.kernel,
        out_type=jax.ShapeDtypeStruct((_L, _D // 8, nw, 8, _CH), jnp.float32),
        mesh=mesh,
        compiler_params=pltpu.CompilerParams(use_tc_tiling_on_sc=False,
                                             needs_layout_passes=False),
        scratch_types=[
            pltpu.VMEM((_L, _CH), jnp.int32),
            pltpu.VMEM((_L, _D), jnp.float32),
            pltpu.VMEM((_CH, _D), jnp.float32),
            pltpu.VMEM((_CH, _D), jnp.float32),
            pltpu.VMEM((_D // 8, 8, _CH + 1), jnp.float32),
            pltpu.VMEM((_D // 8, 8, _CH + 1), jnp.float32),
            pltpu.SemaphoreType.DMA,
            pltpu.SemaphoreType.DMA,
            pltpu.SemaphoreType.DMA,
            pltpu.SemaphoreType.DMA,
        ],
    )
    def emb_kernel(tok_hbm, xt_hbm, pos_hbm, out_hbm,
                   idx_v, pos_v, gbuf0, gbuf1, tbuf0, tbuf1,
                   gsem0, gsem1, ssem0, ssem1):
        wid = lax.axis_index("s") * nc + lax.axis_index("c")

        pltpu.sync_copy(xt_hbm.at[:, pl.ds(wid * _CH, _CH)], idx_v)
        pltpu.sync_copy(pos_hbm, pos_v)

        pltpu.async_copy(tok_hbm.at[idx_v.at[0]], gbuf0, gsem0)

        lane = lax.iota(jnp.int32, 16)
        zero = lane * 0
        lo0, lo1 = lane >> 3, lane & 7
        hi = lane + 16
        hi0, hi1 = hi >> 3, hi & 7

        def step(g, gbuf_b, gsem_b, ssem_b, tbuf_b, gbuf_n, gsem_n, ssem_n,
                 tbuf_n):
            @pl.when(g >= 1)
            def _():
                pltpu.make_async_copy(
                    tbuf_n.at[:, :, pl.ds(0, _CH)],
                    out_hbm.at[g - 1, :, wid], ssem_n).wait()

            @pl.when(g + 1 < _L)
            def _():
                pltpu.async_copy(tok_hbm.at[idx_v.at[g + 1]], gbuf_n, gsem_n)

            pltpu.make_async_copy(tok_hbm.at[idx_v.at[g]], gbuf_b, gsem_b).wait()

            pos_lo = pos_v[g, pl.ds(0, 16)]
            pos_hi = pos_v[g, pl.ds(16, 16)]
            for b0 in range(0, _CH, 8):
                vlo = [gbuf_b[b0 + k, pl.ds(0, 16)] + pos_lo for k in range(8)]
                vhi = [gbuf_b[b0 + k, pl.ds(16, 16)] + pos_hi for k in range(8)]
                for k in range(8):
                    cb = zero + (b0 + k)
                    plsc.store_scatter(tbuf_b, [lo0, lo1, cb], vlo[k])
                    plsc.store_scatter(tbuf_b, [hi0, hi1, cb], vhi[k])

            pltpu.async_copy(tbuf_b.at[:, :, pl.ds(0, _CH)],
                             out_hbm.at[g, :, wid], ssem_b)

        def outer(i, carry):
            g = i * 2
            step(g, gbuf0, gsem0, ssem0, tbuf0, gbuf1, gsem1, ssem1, tbuf1)
            step(g + 1, gbuf1, gsem1, ssem1, tbuf1, gbuf0, gsem0, ssem0, tbuf0)
            return carry

        lax.fori_loop(0, _L // 2, outer, 0)

        pltpu.make_async_copy(tbuf1.at[:, :, pl.ds(0, _CH)],
                              out_hbm.at[_L - 1, :, wid], ssem1).wait()

    return emb_kernel, nw


_V = 1000000
_TBC = 4096
_TGRID = (_V + _TBC - 1) // _TBC


def _make_tc_transpose():
    def body(in_ref, out_ref):
        z = in_ref[...].T.reshape(_TBC // 4, 4, _D)
        out_ref[...] = jnp.concatenate([z[:, q, :] for q in range(4)], axis=1)

    return pl.pallas_call(
        body,
        grid=(_TGRID,),
        in_specs=[pl.BlockSpec((_D, _TBC), lambda k: (0, k))],
        out_specs=pl.BlockSpec((_TBC // 4, 4 * _D), lambda k: (k, 0)),
        out_shape=jax.ShapeDtypeStruct((_V // 4, 4 * _D), jnp.float32),
    )


def kernel(x, token_table, pos_table):
    emb, nw = _make_kernel()
    trans = _make_tc_transpose()
    tblT = jnp.transpose(token_table)
    tbl_lin = trans(tblT).reshape(_V, _D)
    xt = jnp.transpose(x.astype(jnp.int32))
    out5 = emb(tbl_lin, xt, pos_table)
    out = jnp.transpose(out5, (2, 4, 0, 1, 3))
    return out.reshape(_B, _L, _D)


# speedup vs baseline: 6.8919x; 1.3192x over previous
import functools

import jax
import jax.numpy as jnp
from jax import lax
from jax.experimental import pallas as pl
from jax.experimental.pallas import tpu as pltpu
from jax.experimental.pallas import tpu_sc as plsc

_B, _L, _D = 4096, 200, 32
_CH = 128


def _make_kernel():
    mesh = plsc.VectorSubcoreMesh(core_axis_name="c", subcore_axis_name="s")
    nc, ns = mesh.num_cores, mesh.num_subcores
    nw = nc * ns
    assert _B // _CH == nw

    @functools.partial(
        pl.kernel,
        out_type=jax.ShapeDtypeStruct((_L, _D // 8, nw, 8, _CH), jnp.float32),
        mesh=mesh,
        compiler_params=pltpu.CompilerParams(use_tc_tiling_on_sc=False,
                                             needs_layout_passes=False),
        scratch_types=[
            pltpu.VMEM((_L, _CH), jnp.int32),
            pltpu.VMEM((_L, _D), jnp.float32),
            pltpu.VMEM((_CH, _D), jnp.float32),
            pltpu.VMEM((_CH, _D), jnp.float32),
            pltpu.VMEM((_D // 8, 8, _CH + 1), jnp.float32),
            pltpu.VMEM((_D // 8, 8, _CH + 1), jnp.float32),
            pltpu.SemaphoreType.DMA,
            pltpu.SemaphoreType.DMA,
            pltpu.SemaphoreType.DMA,
            pltpu.SemaphoreType.DMA,
        ],
    )
    def emb_kernel(tok_hbm, xt_hbm, pos_hbm, out_hbm,
                   idx_v, pos_v, gbuf0, gbuf1, tbuf0, tbuf1,
                   gsem0, gsem1, ssem0, ssem1):
        wid = lax.axis_index("s") * nc + lax.axis_index("c")

        pltpu.sync_copy(xt_hbm.at[:, pl.ds(wid * _CH, _CH)], idx_v)
        pltpu.sync_copy(pos_hbm, pos_v)

        pltpu.async_copy(tok_hbm.at[idx_v.at[0]], gbuf0, gsem0)

        lane = lax.iota(jnp.int32, 16)
        zero = lane * 0
        lo0, lo1 = lane >> 3, lane & 7
        hi = lane + 16
        hi0, hi1 = hi >> 3, hi & 7

        def step(g, gbuf_b, gsem_b, ssem_b, tbuf_b, gbuf_n, gsem_n, ssem_n,
                 tbuf_n):
            @pl.when(g >= 1)
            def _():
                pltpu.make_async_copy(
                    tbuf_n.at[:, :, pl.ds(0, _CH)],
                    out_hbm.at[g - 1, :, wid], ssem_n).wait()

            @pl.when(g + 1 < _L)
            def _():
                pltpu.async_copy(tok_hbm.at[idx_v.at[g + 1]], gbuf_n, gsem_n)

            pltpu.make_async_copy(tok_hbm.at[idx_v.at[g]], gbuf_b, gsem_b).wait()

            pos_lo = pos_v[g, pl.ds(0, 16)]
            pos_hi = pos_v[g, pl.ds(16, 16)]
            for b0 in range(0, _CH, 8):
                vlo = [gbuf_b[b0 + k, pl.ds(0, 16)] + pos_lo for k in range(8)]
                vhi = [gbuf_b[b0 + k, pl.ds(16, 16)] + pos_hi for k in range(8)]
                for k in range(8):
                    cb = zero + (b0 + k)
                    plsc.store_scatter(tbuf_b, [lo0, lo1, cb], vlo[k])
                    plsc.store_scatter(tbuf_b, [hi0, hi1, cb], vhi[k])

            pltpu.async_copy(tbuf_b.at[:, :, pl.ds(0, _CH)],
                             out_hbm.at[g, :, wid], ssem_b)

        def outer(i, carry):
            g = i * 2
            step(g, gbuf0, gsem0, ssem0, tbuf0, gbuf1, gsem1, ssem1, tbuf1)
            step(g + 1, gbuf1, gsem1, ssem1, tbuf1, gbuf0, gsem0, ssem0, tbuf0)
            return carry

        lax.fori_loop(0, _L // 2, outer, 0)

        pltpu.make_async_copy(tbuf1.at[:, :, pl.ds(0, _CH)],
                              out_hbm.at[_L - 1, :, wid], ssem1).wait()

    return emb_kernel, nw


_V = 1000000
_TBC = 8192
_TGRID = (_V + _TBC - 1) // _TBC


def _make_tc_transpose():
    def body(in_ref, out_ref):
        for q in range(4):
            out_ref[:, 32 * q:32 * (q + 1)] = \
                in_ref[:, pl.ds(q * (_TBC // 4), _TBC // 4)][...].T

    return pl.pallas_call(
        body,
        grid=(_TGRID,),
        in_specs=[pl.BlockSpec((_D, _TBC), lambda k: (0, k))],
        out_specs=pl.BlockSpec((_TBC // 4, 4 * _D), lambda k: (k, 0)),
        out_shape=jax.ShapeDtypeStruct((_TGRID * _TBC // 4, 4 * _D),
                                       jnp.float32),
    )


def kernel(x, token_table, pos_table):
    emb, nw = _make_kernel()
    trans = _make_tc_transpose()
    tblT = jnp.transpose(token_table)
    tbl_lin = trans(tblT).reshape(_TGRID * _TBC, _D)
    xi = x.astype(jnp.int32)
    quarter = _TBC // 4
    xs = ((xi & ~(_TBC - 1)) | ((xi & (quarter - 1)) << 2)
          | ((xi // quarter) & 3))
    xt = jnp.transpose(xs)
    out5 = emb(tbl_lin, xt, pos_table)
    out = jnp.transpose(out5, (2, 4, 0, 1, 3))
    return out.reshape(_B, _L, _D)
